# baseline (speedup 1.0000x reference)
import functools

import jax
import jax.numpy as jnp
from jax import lax
from jax.experimental import pallas as pl
from jax.experimental.pallas import tpu as pltpu
from jax.experimental.pallas import tpu_sc as plsc

B = 16384
NW = 32
BPW = B // NW
CH = 128
NCH = BPW // CH
BLK = 2048
GROWS_G = 62500
GROWS_M = 125000


def _sc_gather(ug, ig, um, im, gut, git, mut, mit):
    mesh = plsc.VectorSubcoreMesh(core_axis_name="c", subcore_axis_name="s")

    @functools.partial(
        pl.kernel,
        mesh=mesh,
        out_type=[jax.ShapeDtypeStruct((B, 128), jnp.float32)] * 4,
        scratch_types=[
            pltpu.VMEM((NCH, CH), jnp.int32),
            pltpu.VMEM((NCH, CH), jnp.int32),
            pltpu.VMEM((NCH, CH), jnp.int32),
            pltpu.VMEM((NCH, CH), jnp.int32),
            pltpu.VMEM((CH, 128), jnp.float32),
            pltpu.VMEM((CH, 128), jnp.float32),
            pltpu.VMEM((CH, 128), jnp.float32),
            pltpu.VMEM((CH, 128), jnp.float32),
            pltpu.SemaphoreType.DMA,
            pltpu.SemaphoreType.DMA,
        ],
    )
    def k(ug_h, ig_h, um_h, im_h, gut_h, git_h, mut_h, mit_h,
          gu_o, gi_o, mu_o, mi_o,
          bug, big, bum, bim, bgu, bgi, bmu, bmi, semg, semw):
        wid = lax.axis_index("s") * 2 + lax.axis_index("c")
        base = wid * BPW
        pltpu.sync_copy(ug_h.at[wid], bug)
        pltpu.sync_copy(ig_h.at[wid], big)
        pltpu.sync_copy(um_h.at[wid], bum)
        pltpu.sync_copy(im_h.at[wid], bim)
        writes = []
        for c in range(NCH):
            gs = [
                pltpu.async_copy(gut_h.at[bug.at[c]], bgu, semg),
                pltpu.async_copy(git_h.at[big.at[c]], bgi, semg),
                pltpu.async_copy(mut_h.at[bum.at[c]], bmu, semg),
                pltpu.async_copy(mit_h.at[bim.at[c]], bmi, semg),
            ]
            for gcp in gs:
                gcp.wait()
            dst = pl.ds(base + c * CH, CH)
            writes.append(pltpu.async_copy(bgu, gu_o.at[dst], semw))
            writes.append(pltpu.async_copy(bgi, gi_o.at[dst], semw))
            writes.append(pltpu.async_copy(bmu, mu_o.at[dst], semw))
            writes.append(pltpu.async_copy(bmi, mi_o.at[dst], semw))
            if c + 1 < NCH:
                for w in writes[-4:]:
                    w.wait()
        for w in writes[-4:]:
            w.wait()

    return k(ug, ig, um, im, gut, git, mut, mit)


def _tc_body(GU, GI, MU, MI, S, W1, b1, W2, b2, Wp, bp, out):
    f32 = jnp.float32
    ci = lax.broadcasted_iota(jnp.int32, (BLK, 128), 1)
    s = S[...]
    mU8 = (ci >> 3 == s[:, 0:1]).astype(f32)
    mI8 = (ci >> 3 == s[:, 1:2]).astype(f32)
    mU16 = (ci >> 4 == s[:, 2:3]).astype(f32)
    mI16 = (ci >> 4 == s[:, 3:4]).astype(f32)

    ri = lax.broadcasted_iota(jnp.int32, (128, 8), 0)
    kj = lax.broadcasted_iota(jnp.int32, (128, 8), 1)
    P8 = ((ri & 7) == kj).astype(f32)

    w1 = W1[...]
    w1a = jnp.concatenate([w1[:16]] * 8, axis=0)
    w1b = jnp.concatenate([w1[16:]] * 8, axis=0)

    dot = functools.partial(jnp.dot, preferred_element_type=f32)
    h = jnp.maximum(
        dot(MU[...] * mU16, w1a) + dot(MI[...] * mI16, w1b) + b1[...], 0.0)
    m = jnp.maximum(dot(h, W2[...]) + b2[...], 0.0)
    g = dot(GU[...] * mU8, P8) * dot(GI[...] * mI8, P8)
    wp = Wp[...]
    val = dot(g, wp[:8, :]) + dot(m, wp[8:, :]) + bp[...]
    out[...] = jax.nn.sigmoid(val)


def _tc_dense(GU, GI, MU, MI, S, W1, b1, W2, b2, Wp, bp):
    grid = B // BLK
    return pl.pallas_call(
        _tc_body,
        grid=(grid,),
        in_specs=[
            pl.BlockSpec((BLK, 128), lambda i: (i, 0)),
            pl.BlockSpec((BLK, 128), lambda i: (i, 0)),
            pl.BlockSpec((BLK, 128), lambda i: (i, 0)),
            pl.BlockSpec((BLK, 128), lambda i: (i, 0)),
            pl.BlockSpec((BLK, 4), lambda i: (i, 0)),
            pl.BlockSpec((32, 16), lambda i: (0, 0)),
            pl.BlockSpec((1, 16), lambda i: (0, 0)),
            pl.BlockSpec((16, 8), lambda i: (0, 0)),
            pl.BlockSpec((1, 8), lambda i: (0, 0)),
            pl.BlockSpec((16, 1), lambda i: (0, 0)),
            pl.BlockSpec((1, 1), lambda i: (0, 0)),
        ],
        out_specs=pl.BlockSpec((BLK, 1), lambda i: (i, 0)),
        out_shape=jax.ShapeDtypeStruct((B, 1), jnp.float32),
    )(GU, GI, MU, MI, S, W1, b1, W2, b2, Wp, bp)


def kernel(user, item, gmf_user_emb, gmf_item_emb, mlp_user_emb, mlp_item_emb,
           W1, b1, W2, b2, Wp, bp):
    u = user.astype(jnp.int32)
    v = item.astype(jnp.int32)
    ug = (u >> 4).reshape(NW, NCH, CH)
    ig = (v >> 4).reshape(NW, NCH, CH)
    um = (u >> 3).reshape(NW, NCH, CH)
    im = (v >> 3).reshape(NW, NCH, CH)
    S = jnp.stack([u & 15, v & 15, u & 7, v & 7], axis=1)
    gut = gmf_user_emb.reshape(GROWS_G, 128)
    git = gmf_item_emb.reshape(GROWS_G, 128)
    mut = mlp_user_emb.reshape(GROWS_M, 128)
    mit = mlp_item_emb.reshape(GROWS_M, 128)
    GU, GI, MU, MI = _sc_gather(ug, ig, um, im, gut, git, mut, mit)
    out = _tc_dense(GU, GI, MU, MI, S,
                    W1, b1.reshape(1, 16), W2, b2.reshape(1, 8),
                    Wp, bp.reshape(1, 1))
    return out.reshape(-1)


# speedup vs baseline: 8.2953x; 8.2953x over previous
import functools

import jax
import jax.numpy as jnp
from jax import lax
from jax.experimental import pallas as pl
from jax.experimental.pallas import tpu as pltpu
from jax.experimental.pallas import tpu_sc as plsc

B = 16384
NW = 32
BPW = B // NW
CH = 16
NCH = BPW // CH
BLK = 2048


def _sc_gather(user_rs, item_rs, guT, giT, muT, miT):
    mesh = plsc.VectorSubcoreMesh(core_axis_name="c", subcore_axis_name="s")

    @functools.partial(
        pl.kernel,
        mesh=mesh,
        compiler_params=pltpu.CompilerParams(needs_layout_passes=False),
        out_type=[
            jax.ShapeDtypeStruct((8, B), jnp.float32),
            jax.ShapeDtypeStruct((8, B), jnp.float32),
            jax.ShapeDtypeStruct((16, B), jnp.float32),
            jax.ShapeDtypeStruct((16, B), jnp.float32),
        ],
        scratch_types=[
            pltpu.VMEM((BPW,), jnp.int32),
            pltpu.VMEM((BPW,), jnp.int32),
            pltpu.VMEM((CH, 8, 128), jnp.float32),
            pltpu.VMEM((CH, 8, 128), jnp.float32),
            pltpu.VMEM((CH, 16, 128), jnp.float32),
            pltpu.VMEM((CH, 16, 128), jnp.float32),
            pltpu.VMEM((8, BPW), jnp.float32),
            pltpu.VMEM((8, BPW), jnp.float32),
            pltpu.VMEM((16, BPW), jnp.float32),
            pltpu.VMEM((16, BPW), jnp.float32),
            pltpu.SemaphoreType.DMA,
        ],
    )
    def k(user_h, item_h, gu_h, gi_h, mu_h, mi_h,
          gu_o, gi_o, mu_o, mi_o,
          vu, vi, tgu, tgi, tmu, tmi, bgu, bgi, bmu, bmi, sem):
        wid = lax.axis_index("s") * 2 + lax.axis_index("c")
        base = wid * BPW
        pltpu.sync_copy(user_h.at[wid], vu)
        pltpu.sync_copy(item_h.at[wid], vi)
        jvec = lax.iota(jnp.int32, 16)

        def chunk(c, carry):
            p0 = c * CH
            sl = pl.ds(p0, CH)
            uvals = vu[sl]
            ivals = vi[sl]
            descs = []
            for j in range(CH):
                tu = pl.multiple_of((uvals[j] >> 7) * 128, 128)
                ti = pl.multiple_of((ivals[j] >> 7) * 128, 128)
                descs.append(pltpu.async_copy(
                    gu_h.at[:, pl.ds(tu, 128)], tgu.at[j], sem))
                descs.append(pltpu.async_copy(
                    gi_h.at[:, pl.ds(ti, 128)], tgi.at[j], sem))
                descs.append(pltpu.async_copy(
                    mu_h.at[:, pl.ds(tu, 128)], tmu.at[j], sem))
                descs.append(pltpu.async_copy(
                    mi_h.at[:, pl.ds(ti, 128)], tmi.at[j], sem))
            for d in descs:
                d.wait()
            lu = uvals & 127
            li = ivals & 127
            for kk in range(8):
                kv = jnp.full((16,), kk, jnp.int32)
                bgu[kk, sl] = plsc.load_gather(tgu, [jvec, kv, lu])
                bgi[kk, sl] = plsc.load_gather(tgi, [jvec, kv, li])
            for kk in range(16):
                kv = jnp.full((16,), kk, jnp.int32)
                bmu[kk, sl] = plsc.load_gather(tmu, [jvec, kv, lu])
                bmi[kk, sl] = plsc.load_gather(tmi, [jvec, kv, li])
            return carry

        lax.fori_loop(0, NCH, chunk, 0)
        dst = pl.ds(base, BPW)
        pltpu.sync_copy(bgu, gu_o.at[:, dst])
        pltpu.sync_copy(bgi, gi_o.at[:, dst])
        pltpu.sync_copy(bmu, mu_o.at[:, dst])
        pltpu.sync_copy(bmi, mi_o.at[:, dst])

    return k(user_rs, item_rs, guT, giT, muT, miT)


def _tc_body(GU, GI, MU, MI, w1a, w1b, b1, w2, b2, wpg, wpm, bp, out):
    dot = functools.partial(jnp.dot, preferred_element_type=jnp.float32)
    h = jnp.maximum(dot(w1a[...], MU[...]) + dot(w1b[...], MI[...]) + b1[...],
                    0.0)
    m = jnp.maximum(dot(w2[...], h) + b2[...], 0.0)
    g = GU[...] * GI[...]
    val = dot(wpg[...], g) + dot(wpm[...], m) + bp[...]
    out[...] = jax.nn.sigmoid(val)


def _tc_dense(GU, GI, MU, MI, w1a, w1b, b1, w2, b2, wpg, wpm, bp):
    grid = B // BLK
    return pl.pallas_call(
        _tc_body,
        grid=(grid,),
        in_specs=[
            pl.BlockSpec((8, BLK), lambda i: (0, i)),
            pl.BlockSpec((8, BLK), lambda i: (0, i)),
            pl.BlockSpec((16, BLK), lambda i: (0, i)),
            pl.BlockSpec((16, BLK), lambda i: (0, i)),
            pl.BlockSpec((16, 16), lambda i: (0, 0)),
            pl.BlockSpec((16, 16), lambda i: (0, 0)),
            pl.BlockSpec((16, 1), lambda i: (0, 0)),
            pl.BlockSpec((8, 16), lambda i: (0, 0)),
            pl.BlockSpec((8, 1), lambda i: (0, 0)),
            pl.BlockSpec((1, 8), lambda i: (0, 0)),
            pl.BlockSpec((1, 8), lambda i: (0, 0)),
            pl.BlockSpec((1, 1), lambda i: (0, 0)),
        ],
        out_specs=pl.BlockSpec((1, BLK), lambda i: (0, i)),
        out_shape=jax.ShapeDtypeStruct((1, B), jnp.float32),
    )(GU, GI, MU, MI, w1a, w1b, b1, w2, b2, wpg, wpm, bp)


def kernel(user, item, gmf_user_emb, gmf_item_emb, mlp_user_emb, mlp_item_emb,
           W1, b1, W2, b2, Wp, bp):
    user_rs = user.astype(jnp.int32).reshape(NW, BPW)
    item_rs = item.astype(jnp.int32).reshape(NW, BPW)
    GU, GI, MU, MI = _sc_gather(
        user_rs, item_rs,
        gmf_user_emb.T, gmf_item_emb.T, mlp_user_emb.T, mlp_item_emb.T)
    out = _tc_dense(
        GU, GI, MU, MI,
        W1[:16].T, W1[16:].T, b1.reshape(16, 1),
        W2.T, b2.reshape(8, 1),
        Wp[:8].T, Wp[8:].T, bp.reshape(1, 1))
    return out.reshape(-1)
